# baseline (speedup 1.0000x reference)
import functools

import jax
import jax.numpy as jnp
from jax.experimental import pallas as pl
from jax.experimental.pallas import tpu as pltpu
from jax.experimental.pallas import tpu_sc as plsc

_K = 8
_BLK = 1024
_TILE = 256
_GRP = 1


def _logits_t_block(x_ref, w_ref, out_ref):
    out_ref[...] = jax.lax.dot_general(
        w_ref[...].astype(jnp.bfloat16), x_ref[...].astype(jnp.bfloat16),
        (((1,), (1,)), ((), ())),
        preferred_element_type=jnp.float32,
        precision=jax.lax.Precision.DEFAULT)


def _tc_logits_t(x, W):
    b, d = x.shape
    e = W.shape[0]
    return pl.pallas_call(
        _logits_t_block,
        grid=(b // _BLK,),
        in_specs=[
            pl.BlockSpec((_BLK, d), lambda i: (i, 0)),
            pl.BlockSpec((e, d), lambda i: (0, 0)),
        ],
        out_specs=pl.BlockSpec((e, _BLK), lambda i: (0, i)),
        out_shape=jax.ShapeDtypeStruct((e, b), jnp.float32),
        compiler_params=pltpu.CompilerParams(
            dimension_semantics=("arbitrary",)),
    )(x, W)


def _sc_tail_t(logits_t):
    e, b = logits_t.shape
    info = plsc.get_sparse_core_info()
    nc, ns = info.num_cores, info.num_subcores
    nw = nc * ns
    rows_w = b // nw
    n_tiles = rows_w // _TILE
    n_iters = _TILE // (16 * _GRP)
    mesh = plsc.VectorSubcoreMesh(core_axis_name="c", subcore_axis_name="s")

    @functools.partial(
        pl.kernel, mesh=mesh,
        out_type=[
            jax.ShapeDtypeStruct((e, b), jnp.float32),
            jax.ShapeDtypeStruct((_K, b), jnp.int32),
        ],
        scratch_types=[
            pltpu.VMEM((e, _TILE), jnp.float32),
            pltpu.VMEM((e, _TILE), jnp.float32),
            pltpu.VMEM((_K, _TILE), jnp.int32),
        ],
    )
    def tail(lt_hbm, fwt_hbm, idxt_hbm, in_v, fw_v, idx_v):
        cid = jax.lax.axis_index("c")
        sid = jax.lax.axis_index("s")
        wid = sid * nc + cid
        base = wid * rows_w

        def one_group(c0):
            rv = [jnp.full((16,), -jnp.inf, jnp.float32) for _ in range(_K)]
            ri = [jnp.full((16,), float(e), jnp.float32) for _ in range(_K)]
            for ee in range(e):
                cv = in_v[ee, pl.ds(c0, 16)]
                ci = jnp.full((16,), float(ee), jnp.float32)
                for k in range(_K):
                    gt = cv > rv[k]
                    rv[k], cv = (jnp.where(gt, cv, rv[k]),
                                 jnp.where(gt, rv[k], cv))
                    ri[k], ci = (jnp.where(gt, ci, ri[k]),
                                 jnp.where(gt, ri[k], ci))
            m0, t8, i7 = rv[0], rv[_K - 1], ri[_K - 1]
            z = jnp.exp(rv[0] - m0)
            for k in range(1, _K):
                z = z + jnp.exp(rv[k] - m0)
            rz = 1.0 / z
            for k in range(_K):
                idx_v[k, pl.ds(c0, 16)] = ri[k].astype(jnp.int32)
            for ee in range(e):
                v = in_v[ee, pl.ds(c0, 16)]
                ef = jnp.full((16,), float(ee), jnp.float32)
                sel = (v > t8) | ((v == t8) & (ef <= i7))
                fw_v[ee, pl.ds(c0, 16)] = jnp.where(
                    sel, jnp.exp(v - m0) * rz, 0.0)

        def iter_body(it, carry):
            for gg in range(_GRP):
                one_group((it * _GRP + gg) * 16)
            return carry

        for t in range(n_tiles):
            r0 = base + t * _TILE
            pltpu.sync_copy(lt_hbm.at[:, pl.ds(r0, _TILE)], in_v)
            jax.lax.fori_loop(0, n_iters, iter_body, 0)
            pltpu.sync_copy(fw_v, fwt_hbm.at[:, pl.ds(r0, _TILE)])
            pltpu.sync_copy(idx_v, idxt_hbm.at[:, pl.ds(r0, _TILE)])

    return tail(logits_t)


def kernel(x, W):
    logits_t = _tc_logits_t(x, W)
    fw_t, idx_t = _sc_tail_t(logits_t)
    return fw_t.T, idx_t.T


# speedup vs baseline: 1.3949x; 1.3949x over previous
import jax
import jax.numpy as jnp
from jax.experimental import pallas as pl
from jax.experimental.pallas import tpu as pltpu

_K = 8
_BLK = 1024


def _router_block(x_ref, w_ref, fw_ref, idx_ref):
    blk, e = fw_ref.shape
    logits = jax.lax.dot_general(
        x_ref[...].astype(jnp.bfloat16), w_ref[...],
        (((1,), (1,)), ((), ())),
        preferred_element_type=jnp.float32,
        precision=jax.lax.Precision.DEFAULT)
    iota = jax.lax.broadcasted_iota(
        jnp.int32, (blk, e), 1).astype(jnp.float32)
    cur = logits
    idx_cols = []
    m0 = None
    for k in range(_K):
        m = jnp.max(cur, axis=1, keepdims=True)
        if k == 0:
            m0 = m
        amax = jnp.min(jnp.where(cur == m, iota, float(e)), axis=1,
                       keepdims=True)
        idx_cols.append(amax)
        cur = jnp.where(iota == amax, -jnp.inf, cur)
    sel = cur == -jnp.inf
    ex = jnp.where(sel, jnp.exp(logits - m0), 0.0)
    z = jnp.sum(ex, axis=1, keepdims=True)
    fw_ref[...] = ex / z
    idx_ref[...] = jnp.concatenate(idx_cols, axis=1).astype(jnp.int32)


def kernel(x, W):
    b, d = x.shape
    e = W.shape[0]
    fw, idx = pl.pallas_call(
        _router_block,
        grid=(b // _BLK,),
        in_specs=[
            pl.BlockSpec((_BLK, d), lambda i: (i, 0)),
            pl.BlockSpec((e, d), lambda i: (0, 0)),
        ],
        out_specs=[
            pl.BlockSpec((_BLK, e), lambda i: (i, 0)),
            pl.BlockSpec((_BLK, _K), lambda i: (i, 0)),
        ],
        out_shape=[
            jax.ShapeDtypeStruct((b, e), jnp.float32),
            jax.ShapeDtypeStruct((b, _K), jnp.int32),
        ],
        compiler_params=pltpu.CompilerParams(
            dimension_semantics=("parallel",)),
    )(x, W.astype(jnp.bfloat16))
    return fw, idx


# speedup vs baseline: 1.4099x; 1.0108x over previous
import jax
import jax.numpy as jnp
from jax.experimental import pallas as pl
from jax.experimental.pallas import tpu as pltpu

_K = 8
_BLK = 1024


def _router_block(x_ref, w_ref, fw_ref, idx_ref):
    blk, e = fw_ref.shape
    logits = jax.lax.dot_general(
        x_ref[...].astype(jnp.bfloat16), w_ref[...].astype(jnp.bfloat16),
        (((1,), (1,)), ((), ())),
        preferred_element_type=jnp.float32,
        precision=jax.lax.Precision.DEFAULT)
    iota = jax.lax.broadcasted_iota(
        jnp.int32, (blk, e), 1).astype(jnp.float32)
    cur = logits
    idx_cols = []
    m0 = None
    for k in range(_K):
        m = jnp.max(cur, axis=1, keepdims=True)
        if k == 0:
            m0 = m
        amax = jnp.min(jnp.where(cur == m, iota, float(e)), axis=1,
                       keepdims=True)
        idx_cols.append(amax)
        cur = jnp.where(iota == amax, -jnp.inf, cur)
    sel = cur == -jnp.inf
    ex = jnp.where(sel, jnp.exp(logits - m0), 0.0)
    z = jnp.sum(ex, axis=1, keepdims=True)
    fw_ref[...] = ex / z
    idx_ref[...] = jnp.concatenate(idx_cols, axis=1).astype(jnp.int32)


def kernel(x, W):
    b, d = x.shape
    e = W.shape[0]
    fw, idx = pl.pallas_call(
        _router_block,
        grid=(b // _BLK,),
        in_specs=[
            pl.BlockSpec((_BLK, d), lambda i: (i, 0)),
            pl.BlockSpec((e, d), lambda i: (0, 0)),
        ],
        out_specs=[
            pl.BlockSpec((_BLK, e), lambda i: (i, 0)),
            pl.BlockSpec((_BLK, _K), lambda i: (i, 0)),
        ],
        out_shape=[
            jax.ShapeDtypeStruct((b, e), jnp.float32),
            jax.ShapeDtypeStruct((b, _K), jnp.int32),
        ],
        compiler_params=pltpu.CompilerParams(
            dimension_semantics=("parallel",)),
    )(x, W)
    return fw, idx
